# Initial kernel scaffold; baseline (speedup 1.0000x reference)
import jax
import jax.numpy as jnp
from jax.experimental import pallas as pl


def kernel(x, edge_index, W_l1, W_r1, b1, W_l2, W_r2, b2):
    raise NotImplementedError("write your pallas kernel here")


# speedup vs baseline: 2.7369x; 2.7369x over previous
import functools

import jax
import jax.numpy as jnp
from jax import lax
from jax.experimental import pallas as pl
from jax.experimental.pallas import tpu as pltpu
from jax.experimental.pallas import tpu_sc as plsc

_N = 10000
_E = 320000
_D = 128
_DH = 64
_NC = 2
_NS = 16
_NW = _NC * _NS
_NP = 10240
_EROWS = 2560
_EPAD = _EROWS * 128
_RPT = _EROWS // _NW
_K = 4
_NCHUNK = _RPT // _K
_NPS = _NP // _NS


def _make_sc_segsum(with_counts: bool):
    mesh = plsc.VectorSubcoreMesh(core_axis_name="c", subcore_axis_name="s")
    out_type = [jax.ShapeDtypeStruct((_NC * _NP, _DH), jnp.float32)]
    scratch = [
        pltpu.VMEM((_K, 128), jnp.int32),
        pltpu.VMEM((_K, 128), jnp.int32),
        pltpu.VMEM((_K * 128, _DH), jnp.float32),
        pltpu.VMEM_SHARED((_NP, _DH), jnp.float32),
        pltpu.SemaphoreType.DMA,
    ]
    if with_counts:
        out_type.append(jax.ShapeDtypeStruct((_NC * _NP, 16), jnp.float32))
        scratch += [
            pltpu.VMEM((128, 16), jnp.float32),
            pltpu.VMEM_SHARED((_NP, 16), jnp.float32),
        ]

    def body(table, src2d, dst2d, zrow, *rest):
        if with_counts:
            (zcnt, ones_h, sum_out, cnt_out,
             src_v, dst_v, rows_v, acc_sh, sem, ones_v, cnt_sh) = rest
        else:
            (sum_out, src_v, dst_v, rows_v, acc_sh, sem) = rest
        cid = lax.axis_index("c")
        sid = lax.axis_index("s")
        wid = cid * _NS + sid

        pltpu.sync_copy(zrow, acc_sh.at[pl.ds(sid * _NPS, _NPS)])
        if with_counts:
            pltpu.sync_copy(zcnt, cnt_sh.at[pl.ds(sid * _NPS, _NPS)])
            pltpu.sync_copy(ones_h, ones_v)
        plsc.subcore_barrier()

        row_base = wid * _RPT

        def chunk(i, carry):
            r0 = row_base + i * _K
            pltpu.sync_copy(src2d.at[pl.ds(r0, _K)], src_v)
            pltpu.sync_copy(dst2d.at[pl.ds(r0, _K)], dst_v)
            cps = [
                pltpu.async_copy(table.at[src_v.at[j]],
                                 rows_v.at[pl.ds(j * 128, 128)], sem)
                for j in range(_K)
            ]
            for cp in cps:
                cp.wait()
            for j in range(_K):
                pltpu.sync_copy(rows_v.at[pl.ds(j * 128, 128)],
                                acc_sh.at[dst_v.at[j]], add=True)
                if with_counts:
                    pltpu.sync_copy(ones_v, cnt_sh.at[dst_v.at[j]], add=True)
            return carry

        lax.fori_loop(0, _NCHUNK, chunk, 0)
        plsc.subcore_barrier()

        obase = cid * _NP + sid * _NPS
        pltpu.sync_copy(acc_sh.at[pl.ds(sid * _NPS, _NPS)],
                        sum_out.at[pl.ds(obase, _NPS)])
        if with_counts:
            pltpu.sync_copy(cnt_sh.at[pl.ds(sid * _NPS, _NPS)],
                            cnt_out.at[pl.ds(obase, _NPS)])

    return pl.kernel(
        body, mesh=mesh, out_type=out_type, scratch_types=scratch,
        compiler_params=pltpu.CompilerParams(use_tc_tiling_on_sc=False))


_sc_segsum_counts = _make_sc_segsum(True)
_sc_segsum_plain = _make_sc_segsum(False)


def _dense_body(relu, split_out, p0_ref, p1_ref, cnt_ref, x0_ref, x1_ref,
                wl_ref, wr_ref, b_ref, *o_refs):
    s = jnp.concatenate([p0_ref[0] + p0_ref[1], p1_ref[0] + p1_ref[1]],
                        axis=1)
    c = cnt_ref[0, :, 0:1] + cnt_ref[1, :, 0:1]
    aggr = s / jnp.clip(c, 1.0, None)
    xfull = jnp.concatenate([x0_ref[...], x1_ref[...]], axis=1)
    out = (jnp.dot(aggr, wl_ref[...], preferred_element_type=jnp.float32)
           + jnp.dot(xfull, wr_ref[...], preferred_element_type=jnp.float32)
           + b_ref[...])
    if relu:
        out = jnp.maximum(out, 0.0)
    if split_out:
        o_refs[0][...] = out[:, :_DH]
        o_refs[1][...] = out[:, _DH:]
    else:
        o_refs[0][...] = out


def _dense(p0, p1, cnt, x0, x1, wl, wr, b, relu, split_out):
    br = 512
    grid = _NP // br
    p0r = p0.reshape(_NC, _NP, _DH)
    p1r = p1.reshape(_NC, _NP, _DH)
    cnt3 = cnt.reshape(_NC, _NP, 16)
    if split_out:
        out_shape = [jax.ShapeDtypeStruct((_NP, _DH), jnp.float32)] * 2
        out_specs = [pl.BlockSpec((br, _DH), lambda i: (i, 0))] * 2
    else:
        out_shape = [jax.ShapeDtypeStruct((_NP, _D), jnp.float32)]
        out_specs = [pl.BlockSpec((br, _D), lambda i: (i, 0))]
    return pl.pallas_call(
        functools.partial(_dense_body, relu, split_out),
        grid=(grid,),
        in_specs=[
            pl.BlockSpec((_NC, br, _DH), lambda i: (0, i, 0)),
            pl.BlockSpec((_NC, br, _DH), lambda i: (0, i, 0)),
            pl.BlockSpec((_NC, br, 16), lambda i: (0, i, 0)),
            pl.BlockSpec((br, _DH), lambda i: (i, 0)),
            pl.BlockSpec((br, _DH), lambda i: (i, 0)),
            pl.BlockSpec((_D, _D), lambda i: (0, 0)),
            pl.BlockSpec((_D, _D), lambda i: (0, 0)),
            pl.BlockSpec((1, _D), lambda i: (0, 0)),
        ],
        out_specs=out_specs,
        out_shape=out_shape,
    )(p0r, p1r, cnt3, x0, x1, wl, wr, b.reshape(1, _D))


def kernel(x, edge_index, W_l1, W_r1, b1, W_l2, W_r2, b2):
    src = edge_index[0].astype(jnp.int32)
    dst = edge_index[1].astype(jnp.int32)
    npad = _EPAD - _E
    src2d = jnp.concatenate(
        [src, jnp.zeros((npad,), jnp.int32)]).reshape(_EROWS, 128)
    dst2d = jnp.concatenate(
        [dst, jnp.full((npad,), _NP - 1, jnp.int32)]).reshape(_EROWS, 128)
    zrow = jnp.zeros((_NPS, _DH), jnp.float32)
    zcnt = jnp.zeros((_NPS, 16), jnp.float32)
    ones = jnp.ones((128, 16), jnp.float32)

    xpad = jnp.concatenate([x, jnp.zeros((_NP - _N, _D), jnp.float32)])
    x0 = xpad[:, :_DH]
    x1 = xpad[:, _DH:]
    s10, cnts = _sc_segsum_counts(x0, src2d, dst2d, zrow, zcnt, ones)
    (s11,) = _sc_segsum_plain(x1, src2d, dst2d, zrow)
    h0, h1 = _dense(s10, s11, cnts, x0, x1, W_l1, W_r1, b1,
                    relu=True, split_out=True)
    (s20,) = _sc_segsum_plain(h0, src2d, dst2d, zrow)
    (s21,) = _sc_segsum_plain(h1, src2d, dst2d, zrow)
    (out,) = _dense(s20, s21, cnts, h0, h1, W_l2, W_r2, b2,
                    relu=False, split_out=False)
    return out[:_N]


# speedup vs baseline: 3.1231x; 1.1411x over previous
import functools

import jax
import jax.numpy as jnp
from jax import lax
from jax.experimental import pallas as pl
from jax.experimental.pallas import tpu as pltpu
from jax.experimental.pallas import tpu_sc as plsc

_N = 10000
_E = 320000
_D = 128
_DH = 64
_NC = 2
_NS = 16
_NW = _NC * _NS
_NP = 10240
_EROWS = 2560
_EPAD = _EROWS * 128
_RPT = _EROWS // _NW
_K = 4
_NCHUNK = _RPT // _K
_T = _NCHUNK // 2
_EROWS_ST = _EROWS + _K
_NPS = _NP // _NS


def _make_sc_segsum(with_counts: bool):
    mesh = plsc.VectorSubcoreMesh(core_axis_name="c", subcore_axis_name="s")
    out_type = [jax.ShapeDtypeStruct((_NC * _NP, _DH), jnp.float32)]
    scratch = [
        pltpu.VMEM((_K, 2, 128), jnp.int32),
        pltpu.VMEM((_K, 2, 128), jnp.int32),
        pltpu.VMEM((_K * 128, _DH), jnp.float32),
        pltpu.VMEM((_K * 128, _DH), jnp.float32),
        pltpu.VMEM_SHARED((_NP, _DH), jnp.float32),
        pltpu.SemaphoreType.DMA,
        pltpu.SemaphoreType.DMA,
        pltpu.SemaphoreType.DMA,
        pltpu.SemaphoreType.DMA,
    ]
    if with_counts:
        out_type.append(jax.ShapeDtypeStruct((_NC * _NP, 16), jnp.float32))
        scratch += [
            pltpu.VMEM((128, 16), jnp.float32),
            pltpu.VMEM((128, 16), jnp.float32),
            pltpu.VMEM_SHARED((_NP, 16), jnp.float32),
        ]

    def body(table, edges, zrow, *rest):
        if with_counts:
            (zcnt, ones_h, sum_out, cnt_out, idx_a, idx_b, rows_a, rows_b,
             acc_sh, g_a, g_b, s_a, s_b, ones_v, zones_v, cnt_sh) = rest
        else:
            (sum_out, idx_a, idx_b, rows_a, rows_b,
             acc_sh, g_a, g_b, s_a, s_b) = rest
        cid = lax.axis_index("c")
        sid = lax.axis_index("s")
        wid = cid * _NS + sid
        idx = (idx_a, idx_b)
        rows = (rows_a, rows_b)
        gsem = (g_a, g_b)
        ssem = (s_a, s_b)

        pltpu.sync_copy(zrow, acc_sh.at[pl.ds(sid * _NPS, _NPS)])
        pltpu.sync_copy(zrow.at[pl.ds(0, _K * 128)], rows_b)
        if with_counts:
            pltpu.sync_copy(zcnt, cnt_sh.at[pl.ds(sid * _NPS, _NPS)])
            pltpu.sync_copy(ones_h, ones_v)
            pltpu.sync_copy(zcnt.at[pl.ds(0, 128)], zones_v)
        plsc.subcore_barrier()

        row_base = wid * _RPT

        def load_idx(buf, chunk):
            pltpu.sync_copy(edges.at[pl.ds(row_base + chunk * _K, _K)],
                            idx[buf])

        def gathers(buf, start):
            for j in range(_K):
                cp = pltpu.make_async_copy(
                    table.at[idx[buf].at[j, 0]],
                    rows[buf].at[pl.ds(j * 128, 128)], gsem[buf])
                if start:
                    cp.start()
                else:
                    cp.wait()

        def scatters(buf, start, prime=False):
            for j in range(_K):
                dsti = idx[buf].at[j, 1]
                cp = pltpu.make_async_copy(
                    rows[buf].at[pl.ds(j * 128, 128)],
                    acc_sh.at[dsti], ssem[buf])
                if start:
                    cp.start(add=True)
                else:
                    cp.wait()
                if with_counts:
                    csrc = zones_v if prime else ones_v
                    cp2 = pltpu.make_async_copy(csrc, cnt_sh.at[dsti],
                                                ssem[buf])
                    if start:
                        cp2.start(add=True)
                    else:
                        cp2.wait()

        load_idx(0, 0)
        gathers(0, start=True)
        load_idx(1, 0)
        scatters(1, start=True, prime=True)

        def pipelined(t, carry):
            i0 = 2 * t
            scatters(1, start=False)
            load_idx(1, i0 + 1)
            gathers(1, start=True)
            gathers(0, start=False)
            scatters(0, start=True)
            scatters(0, start=False)
            load_idx(0, i0 + 2)
            gathers(0, start=True)
            gathers(1, start=False)
            scatters(1, start=True)
            return carry

        lax.fori_loop(0, _T, pipelined, 0)
        gathers(0, start=False)
        scatters(1, start=False)
        plsc.subcore_barrier()

        obase = cid * _NP + sid * _NPS
        pltpu.sync_copy(acc_sh.at[pl.ds(sid * _NPS, _NPS)],
                        sum_out.at[pl.ds(obase, _NPS)])
        if with_counts:
            pltpu.sync_copy(cnt_sh.at[pl.ds(sid * _NPS, _NPS)],
                            cnt_out.at[pl.ds(obase, _NPS)])

    return pl.kernel(
        body, mesh=mesh, out_type=out_type, scratch_types=scratch,
        compiler_params=pltpu.CompilerParams(use_tc_tiling_on_sc=False))


_sc_segsum_counts = _make_sc_segsum(True)
_sc_segsum_plain = _make_sc_segsum(False)


def _dense_body(relu, split_out, p0_ref, p1_ref, cnt_ref, x0_ref, x1_ref,
                wl_ref, wr_ref, b_ref, *o_refs):
    s = jnp.concatenate([p0_ref[0] + p0_ref[1], p1_ref[0] + p1_ref[1]],
                        axis=1)
    c = cnt_ref[0, :, 0:1] + cnt_ref[1, :, 0:1]
    aggr = s / jnp.clip(c, 1.0, None)
    xfull = jnp.concatenate([x0_ref[...], x1_ref[...]], axis=1)
    out = (jnp.dot(aggr, wl_ref[...], preferred_element_type=jnp.float32)
           + jnp.dot(xfull, wr_ref[...], preferred_element_type=jnp.float32)
           + b_ref[...])
    if relu:
        out = jnp.maximum(out, 0.0)
    if split_out:
        o_refs[0][...] = out[:, :_DH]
        o_refs[1][...] = out[:, _DH:]
    else:
        o_refs[0][...] = out


def _dense(p0, p1, cnt, x0, x1, wl, wr, b, relu, split_out):
    br = 512
    grid = _NP // br
    p0r = p0.reshape(_NC, _NP, _DH)
    p1r = p1.reshape(_NC, _NP, _DH)
    cnt3 = cnt.reshape(_NC, _NP, 16)
    if split_out:
        out_shape = [jax.ShapeDtypeStruct((_NP, _DH), jnp.float32)] * 2
        out_specs = [pl.BlockSpec((br, _DH), lambda i: (i, 0))] * 2
    else:
        out_shape = [jax.ShapeDtypeStruct((_NP, _D), jnp.float32)]
        out_specs = [pl.BlockSpec((br, _D), lambda i: (i, 0))]
    return pl.pallas_call(
        functools.partial(_dense_body, relu, split_out),
        grid=(grid,),
        in_specs=[
            pl.BlockSpec((_NC, br, _DH), lambda i: (0, i, 0)),
            pl.BlockSpec((_NC, br, _DH), lambda i: (0, i, 0)),
            pl.BlockSpec((_NC, br, 16), lambda i: (0, i, 0)),
            pl.BlockSpec((br, _DH), lambda i: (i, 0)),
            pl.BlockSpec((br, _DH), lambda i: (i, 0)),
            pl.BlockSpec((_D, _D), lambda i: (0, 0)),
            pl.BlockSpec((_D, _D), lambda i: (0, 0)),
            pl.BlockSpec((1, _D), lambda i: (0, 0)),
        ],
        out_specs=out_specs,
        out_shape=out_shape,
    )(p0r, p1r, cnt3, x0, x1, wl, wr, b.reshape(1, _D))


def kernel(x, edge_index, W_l1, W_r1, b1, W_l2, W_r2, b2):
    src = edge_index[0].astype(jnp.int32)
    dst = edge_index[1].astype(jnp.int32)
    src2d = jnp.concatenate(
        [src, jnp.zeros((_EROWS_ST * 128 - _E,), jnp.int32)]
    ).reshape(_EROWS_ST, 128)
    dst2d = jnp.concatenate(
        [dst, jnp.full((_EROWS_ST * 128 - _E,), _NP - 1, jnp.int32)]
    ).reshape(_EROWS_ST, 128)
    edges = jnp.stack([src2d, dst2d], axis=1)
    zrow = jnp.zeros((_NPS, _DH), jnp.float32)
    zcnt = jnp.zeros((_NPS, 16), jnp.float32)
    ones = jnp.ones((128, 16), jnp.float32)

    xpad = jnp.concatenate([x, jnp.zeros((_NP - _N, _D), jnp.float32)])
    x0 = xpad[:, :_DH]
    x1 = xpad[:, _DH:]
    s10, cnts = _sc_segsum_counts(x0, edges, zrow, zcnt, ones)
    (s11,) = _sc_segsum_plain(x1, edges, zrow)
    h0, h1 = _dense(s10, s11, cnts, x0, x1, W_l1, W_r1, b1,
                    relu=True, split_out=True)
    (s20,) = _sc_segsum_plain(h0, edges, zrow)
    (s21,) = _sc_segsum_plain(h1, edges, zrow)
    (out,) = _dense(s20, s21, cnts, h0, h1, W_l2, W_r2, b2,
                    relu=False, split_out=False)
    return out[:_N]


# speedup vs baseline: 5.2096x; 1.6681x over previous
import functools

import jax
import jax.numpy as jnp
from jax import lax
from jax.experimental import pallas as pl
from jax.experimental.pallas import tpu as pltpu
from jax.experimental.pallas import tpu_sc as plsc

_N = 10000
_E = 320000
_D = 128
_NC = 2
_NS = 16
_NW = _NC * _NS
_NP = 10240
_EROWS = 2560
_RPT = _EROWS // _NW
_K = 4
_NCHUNK = _RPT // _K
_T = _NCHUNK // 2
_EROWS_ST = _EROWS + _K
_NPS = _NP // _NS
_BF = jnp.bfloat16


def _make_sc_segsum(with_counts: bool):
    mesh = plsc.VectorSubcoreMesh(core_axis_name="c", subcore_axis_name="s")
    out_type = [jax.ShapeDtypeStruct((_NC * _NP, _D), _BF)]
    scratch = [
        pltpu.VMEM((_K, 2, 128), jnp.int32),
        pltpu.VMEM((_K, 2, 128), jnp.int32),
        pltpu.VMEM((_K * 128, _D), _BF),
        pltpu.VMEM((_K * 128, _D), _BF),
        pltpu.VMEM_SHARED((_NP, _D), _BF),
        pltpu.SemaphoreType.DMA,
        pltpu.SemaphoreType.DMA,
        pltpu.SemaphoreType.DMA,
        pltpu.SemaphoreType.DMA,
    ]
    if with_counts:
        out_type.append(jax.ShapeDtypeStruct((_NC * _NP, 16), jnp.float32))
        scratch += [
            pltpu.VMEM((128, 16), jnp.float32),
            pltpu.VMEM((128, 16), jnp.float32),
            pltpu.VMEM_SHARED((_NP, 16), jnp.float32),
        ]

    def body(table, edges, zrow, *rest):
        if with_counts:
            (zcnt, ones_h, sum_out, cnt_out, idx_a, idx_b, rows_a, rows_b,
             acc_sh, g_a, g_b, s_a, s_b, ones_v, zones_v, cnt_sh) = rest
        else:
            (sum_out, idx_a, idx_b, rows_a, rows_b,
             acc_sh, g_a, g_b, s_a, s_b) = rest
        cid = lax.axis_index("c")
        sid = lax.axis_index("s")
        wid = cid * _NS + sid
        idx = (idx_a, idx_b)
        rows = (rows_a, rows_b)
        gsem = (g_a, g_b)
        ssem = (s_a, s_b)

        pltpu.sync_copy(zrow, acc_sh.at[pl.ds(sid * _NPS, _NPS)])
        pltpu.sync_copy(zrow.at[pl.ds(0, _K * 128)], rows_b)
        if with_counts:
            pltpu.sync_copy(zcnt, cnt_sh.at[pl.ds(sid * _NPS, _NPS)])
            pltpu.sync_copy(ones_h, ones_v)
            pltpu.sync_copy(zcnt.at[pl.ds(0, 128)], zones_v)
        plsc.subcore_barrier()

        row_base = wid * _RPT

        def load_idx(buf, chunk):
            pltpu.sync_copy(edges.at[pl.ds(row_base + chunk * _K, _K)],
                            idx[buf])

        def gathers(buf, start):
            for j in range(_K):
                cp = pltpu.make_async_copy(
                    table.at[idx[buf].at[j, 0]],
                    rows[buf].at[pl.ds(j * 128, 128)], gsem[buf])
                if start:
                    cp.start()
                else:
                    cp.wait()

        def scatters(buf, start, prime=False):
            for j in range(_K):
                dsti = idx[buf].at[j, 1]
                cp = pltpu.make_async_copy(
                    rows[buf].at[pl.ds(j * 128, 128)],
                    acc_sh.at[dsti], ssem[buf])
                if start:
                    cp.start(add=True)
                else:
                    cp.wait()
                if with_counts:
                    csrc = zones_v if prime else ones_v
                    cp2 = pltpu.make_async_copy(csrc, cnt_sh.at[dsti],
                                                ssem[buf])
                    if start:
                        cp2.start(add=True)
                    else:
                        cp2.wait()

        load_idx(0, 0)
        gathers(0, start=True)
        load_idx(1, 0)
        scatters(1, start=True, prime=True)

        def pipelined(t, carry):
            i0 = 2 * t
            scatters(1, start=False)
            load_idx(1, i0 + 1)
            gathers(1, start=True)
            gathers(0, start=False)
            scatters(0, start=True)
            scatters(0, start=False)
            load_idx(0, i0 + 2)
            gathers(0, start=True)
            gathers(1, start=False)
            scatters(1, start=True)
            return carry

        lax.fori_loop(0, _T, pipelined, 0)
        gathers(0, start=False)
        scatters(1, start=False)
        plsc.subcore_barrier()

        obase = cid * _NP + sid * _NPS
        pltpu.sync_copy(acc_sh.at[pl.ds(sid * _NPS, _NPS)],
                        sum_out.at[pl.ds(obase, _NPS)])
        if with_counts:
            pltpu.sync_copy(cnt_sh.at[pl.ds(sid * _NPS, _NPS)],
                            cnt_out.at[pl.ds(obase, _NPS)])

    return pl.kernel(
        body, mesh=mesh, out_type=out_type, scratch_types=scratch,
        compiler_params=pltpu.CompilerParams(use_tc_tiling_on_sc=False))


_sc_segsum_counts = _make_sc_segsum(True)
_sc_segsum_plain = _make_sc_segsum(False)


def _dense_body(relu, out_bf, p_ref, cnt_ref, x_ref, wl_ref, wr_ref, b_ref,
                o_ref):
    s = p_ref[0].astype(jnp.float32) + p_ref[1].astype(jnp.float32)
    c = cnt_ref[0, :, 0:1] + cnt_ref[1, :, 0:1]
    aggr = s / jnp.clip(c, 1.0, None)
    xf = x_ref[...].astype(jnp.float32)
    out = (jnp.dot(aggr, wl_ref[...], preferred_element_type=jnp.float32)
           + jnp.dot(xf, wr_ref[...], preferred_element_type=jnp.float32)
           + b_ref[...])
    if relu:
        out = jnp.maximum(out, 0.0)
    if out_bf:
        o_ref[...] = out.astype(_BF)
    else:
        o_ref[...] = out


def _dense(p, cnt, xbf, wl, wr, b, relu, out_bf):
    br = 512
    grid = _NP // br
    p3 = p.reshape(_NC, _NP, _D)
    cnt3 = cnt.reshape(_NC, _NP, 16)
    odt = _BF if out_bf else jnp.float32
    return pl.pallas_call(
        functools.partial(_dense_body, relu, out_bf),
        grid=(grid,),
        in_specs=[
            pl.BlockSpec((_NC, br, _D), lambda i: (0, i, 0)),
            pl.BlockSpec((_NC, br, 16), lambda i: (0, i, 0)),
            pl.BlockSpec((br, _D), lambda i: (i, 0)),
            pl.BlockSpec((_D, _D), lambda i: (0, 0)),
            pl.BlockSpec((_D, _D), lambda i: (0, 0)),
            pl.BlockSpec((1, _D), lambda i: (0, 0)),
        ],
        out_specs=pl.BlockSpec((br, _D), lambda i: (i, 0)),
        out_shape=jax.ShapeDtypeStruct((_NP, _D), odt),
    )(p3, cnt3, xbf, wl, wr, b.reshape(1, _D))


def kernel(x, edge_index, W_l1, W_r1, b1, W_l2, W_r2, b2):
    src = edge_index[0].astype(jnp.int32)
    dst = edge_index[1].astype(jnp.int32)
    src2d = jnp.concatenate(
        [src, jnp.zeros((_EROWS_ST * 128 - _E,), jnp.int32)]
    ).reshape(_EROWS_ST, 128)
    dst2d = jnp.concatenate(
        [dst, jnp.full((_EROWS_ST * 128 - _E,), _NP - 1, jnp.int32)]
    ).reshape(_EROWS_ST, 128)
    edges = jnp.stack([src2d, dst2d], axis=1)
    zrow = jnp.zeros((_NPS, _D), _BF)
    zcnt = jnp.zeros((_NPS, 16), jnp.float32)
    ones = jnp.ones((128, 16), jnp.float32)

    xbf = jnp.concatenate(
        [x.astype(_BF), jnp.zeros((_NP - _N, _D), _BF)])
    s1, cnts = _sc_segsum_counts(xbf, edges, zrow, zcnt, ones)
    hbf = _dense(s1, cnts, xbf, W_l1, W_r1, b1, relu=True, out_bf=True)
    (s2,) = _sc_segsum_plain(hbf, edges, zrow)
    out = _dense(s2, cnts, hbf, W_l2, W_r2, b2, relu=False, out_bf=False)
    return out[:_N]


# speedup vs baseline: 9.4221x; 1.8086x over previous
import functools

import jax
import jax.numpy as jnp
from jax import lax
from jax.experimental import pallas as pl
from jax.experimental.pallas import tpu as pltpu
from jax.experimental.pallas import tpu_sc as plsc

_N = 10000
_E = 320000
_D = 128
_NC = 2
_NS = 16
_NW = _NC * _NS
_NP = 10240
_EROWS = 2560
_RPT = _EROWS // _NW
_K = 2
_NCHUNK = _RPT // _K
_T = _NCHUNK // 2
_EROWS_ST = _EROWS + _K
_NPS = _NP // _NS
_BF = jnp.bfloat16


def _make_sc_segsum(with_counts: bool):
    mesh = plsc.VectorSubcoreMesh(core_axis_name="c", subcore_axis_name="s")
    out_type = [jax.ShapeDtypeStruct((_NC * _NP, _D), _BF)]
    scratch = [
        pltpu.VMEM((_K, 2, 128), jnp.int32),
        pltpu.VMEM((_K, 2, 128), jnp.int32),
        pltpu.VMEM((_K * 128, _D), _BF),
        pltpu.VMEM((_K * 128, _D), _BF),
        pltpu.VMEM_SHARED((_NP, _D), _BF),
        pltpu.VMEM_SHARED((_NP, _D), _BF),
        pltpu.SemaphoreType.DMA,
        pltpu.SemaphoreType.DMA,
        pltpu.SemaphoreType.DMA,
        pltpu.SemaphoreType.DMA,
    ]
    if with_counts:
        out_type.append(jax.ShapeDtypeStruct((_NC * _NP, 16), jnp.float32))
        scratch += [
            pltpu.VMEM((128, 16), jnp.float32),
            pltpu.VMEM((128, 16), jnp.float32),
            pltpu.VMEM_SHARED((_NP, 16), jnp.float32),
        ]

    def body(table, edges, zrow, *rest):
        if with_counts:
            (zcnt, ones_h, sum_out, cnt_out, idx_a, idx_b, rows_a, rows_b,
             acc_sh, tab_sh, g_a, g_b, s_a, s_b,
             ones_v, zones_v, cnt_sh) = rest
        else:
            (sum_out, idx_a, idx_b, rows_a, rows_b,
             acc_sh, tab_sh, g_a, g_b, s_a, s_b) = rest
        cid = lax.axis_index("c")
        sid = lax.axis_index("s")
        wid = cid * _NS + sid
        idx = (idx_a, idx_b)
        rows = (rows_a, rows_b)
        gsem = (g_a, g_b)
        ssem = (s_a, s_b)

        pltpu.sync_copy(table.at[pl.ds(sid * _NPS, _NPS)],
                        tab_sh.at[pl.ds(sid * _NPS, _NPS)])
        pltpu.sync_copy(zrow, acc_sh.at[pl.ds(sid * _NPS, _NPS)])
        pltpu.sync_copy(zrow.at[pl.ds(0, _K * 128)], rows_b)
        if with_counts:
            pltpu.sync_copy(zcnt, cnt_sh.at[pl.ds(sid * _NPS, _NPS)])
            pltpu.sync_copy(ones_h, ones_v)
            pltpu.sync_copy(zcnt.at[pl.ds(0, 128)], zones_v)
        plsc.subcore_barrier()

        row_base = wid * _RPT

        def load_idx(buf, chunk):
            pltpu.sync_copy(edges.at[pl.ds(row_base + chunk * _K, _K)],
                            idx[buf])

        def gathers(buf, start):
            for j in range(_K):
                cp = pltpu.make_async_copy(
                    tab_sh.at[idx[buf].at[j, 0]],
                    rows[buf].at[pl.ds(j * 128, 128)], gsem[buf])
                if start:
                    cp.start()
                else:
                    cp.wait()

        def scatters(buf, start, prime=False):
            for j in range(_K):
                dsti = idx[buf].at[j, 1]
                cp = pltpu.make_async_copy(
                    rows[buf].at[pl.ds(j * 128, 128)],
                    acc_sh.at[dsti], ssem[buf])
                if start:
                    cp.start(add=True)
                else:
                    cp.wait()
                if with_counts:
                    csrc = zones_v if prime else ones_v
                    cp2 = pltpu.make_async_copy(csrc, cnt_sh.at[dsti],
                                                ssem[buf])
                    if start:
                        cp2.start(add=True)
                    else:
                        cp2.wait()

        load_idx(0, 0)
        gathers(0, start=True)
        load_idx(1, 0)
        scatters(1, start=True, prime=True)

        def pipelined(t, carry):
            i0 = 2 * t
            scatters(1, start=False)
            load_idx(1, i0 + 1)
            gathers(1, start=True)
            gathers(0, start=False)
            scatters(0, start=True)
            scatters(0, start=False)
            load_idx(0, i0 + 2)
            gathers(0, start=True)
            gathers(1, start=False)
            scatters(1, start=True)
            return carry

        lax.fori_loop(0, _T, pipelined, 0)
        gathers(0, start=False)
        scatters(1, start=False)
        plsc.subcore_barrier()

        obase = cid * _NP + sid * _NPS
        pltpu.sync_copy(acc_sh.at[pl.ds(sid * _NPS, _NPS)],
                        sum_out.at[pl.ds(obase, _NPS)])
        if with_counts:
            pltpu.sync_copy(cnt_sh.at[pl.ds(sid * _NPS, _NPS)],
                            cnt_out.at[pl.ds(obase, _NPS)])

    return pl.kernel(
        body, mesh=mesh, out_type=out_type, scratch_types=scratch,
        compiler_params=pltpu.CompilerParams(use_tc_tiling_on_sc=False))


_sc_segsum_counts = _make_sc_segsum(True)
_sc_segsum_plain = _make_sc_segsum(False)


def _dense_body(relu, out_bf, p_ref, cnt_ref, x_ref, wl_ref, wr_ref, b_ref,
                o_ref):
    s = p_ref[0].astype(jnp.float32) + p_ref[1].astype(jnp.float32)
    c = cnt_ref[0, :, 0:1] + cnt_ref[1, :, 0:1]
    aggr = s / jnp.clip(c, 1.0, None)
    xf = x_ref[...].astype(jnp.float32)
    out = (jnp.dot(aggr, wl_ref[...], preferred_element_type=jnp.float32)
           + jnp.dot(xf, wr_ref[...], preferred_element_type=jnp.float32)
           + b_ref[...])
    if relu:
        out = jnp.maximum(out, 0.0)
    if out_bf:
        o_ref[...] = out.astype(_BF)
    else:
        o_ref[...] = out


def _dense(p, cnt, xbf, wl, wr, b, relu, out_bf):
    br = 512
    grid = _NP // br
    p3 = p.reshape(_NC, _NP, _D)
    cnt3 = cnt.reshape(_NC, _NP, 16)
    odt = _BF if out_bf else jnp.float32
    return pl.pallas_call(
        functools.partial(_dense_body, relu, out_bf),
        grid=(grid,),
        in_specs=[
            pl.BlockSpec((_NC, br, _D), lambda i: (0, i, 0)),
            pl.BlockSpec((_NC, br, 16), lambda i: (0, i, 0)),
            pl.BlockSpec((br, _D), lambda i: (i, 0)),
            pl.BlockSpec((_D, _D), lambda i: (0, 0)),
            pl.BlockSpec((_D, _D), lambda i: (0, 0)),
            pl.BlockSpec((1, _D), lambda i: (0, 0)),
        ],
        out_specs=pl.BlockSpec((br, _D), lambda i: (i, 0)),
        out_shape=jax.ShapeDtypeStruct((_NP, _D), odt),
    )(p3, cnt3, xbf, wl, wr, b.reshape(1, _D))


def kernel(x, edge_index, W_l1, W_r1, b1, W_l2, W_r2, b2):
    src = edge_index[0].astype(jnp.int32)
    dst = edge_index[1].astype(jnp.int32)
    src2d = jnp.concatenate(
        [src, jnp.zeros((_EROWS_ST * 128 - _E,), jnp.int32)]
    ).reshape(_EROWS_ST, 128)
    dst2d = jnp.concatenate(
        [dst, jnp.full((_EROWS_ST * 128 - _E,), _NP - 1, jnp.int32)]
    ).reshape(_EROWS_ST, 128)
    edges = jnp.stack([src2d, dst2d], axis=1)
    zrow = jnp.zeros((_NPS, _D), _BF)
    zcnt = jnp.zeros((_NPS, 16), jnp.float32)
    ones = jnp.ones((128, 16), jnp.float32)

    xbf = jnp.concatenate(
        [x.astype(_BF), jnp.zeros((_NP - _N, _D), _BF)])
    s1, cnts = _sc_segsum_counts(xbf, edges, zrow, zcnt, ones)
    hbf = _dense(s1, cnts, xbf, W_l1, W_r1, b1, relu=True, out_bf=True)
    (s2,) = _sc_segsum_plain(hbf, edges, zrow)
    out = _dense(s2, cnts, hbf, W_l2, W_r2, b2, relu=False, out_bf=False)
    return out[:_N]


# speedup vs baseline: 10.1549x; 1.0778x over previous
import functools

import jax
import jax.numpy as jnp
from jax import lax
from jax.experimental import pallas as pl
from jax.experimental.pallas import tpu as pltpu
from jax.experimental.pallas import tpu_sc as plsc

_N = 10000
_E = 320000
_D = 128
_NC = 2
_NS = 16
_NW = _NC * _NS
_NP = 10240
_EROWS = 2560
_RPT = _EROWS // _NW
_K = 2
_NCHUNK = _RPT // _K
_T = _NCHUNK // 2
_EROWS_ST = _EROWS + _K
_NPS = _NP // _NS
_BF = jnp.bfloat16


def _make_sc_segsum(with_counts: bool):
    mesh = plsc.VectorSubcoreMesh(core_axis_name="c", subcore_axis_name="s")
    out_type = [jax.ShapeDtypeStruct((_NC, _NP, _D), _BF)]
    scratch = [
        pltpu.VMEM((_K, 2, 128), jnp.int32),
        pltpu.VMEM((_K, 2, 128), jnp.int32),
        pltpu.VMEM((_K * 128, _D), _BF),
        pltpu.VMEM((_K * 128, _D), _BF),
        pltpu.VMEM_SHARED((_NP, _D), _BF),
        pltpu.VMEM_SHARED((_NP, _D), _BF),
        pltpu.SemaphoreType.DMA,
        pltpu.SemaphoreType.DMA,
        pltpu.SemaphoreType.DMA,
        pltpu.SemaphoreType.DMA,
    ]
    if with_counts:
        out_type.append(jax.ShapeDtypeStruct((_NC, _NP, 16), jnp.float32))
        scratch += [
            pltpu.VMEM((128, 16), jnp.float32),
            pltpu.VMEM((128, 16), jnp.float32),
            pltpu.VMEM_SHARED((_NP, 16), jnp.float32),
        ]

    def body(table, edges, zrow, *rest):
        if with_counts:
            (zcnt, ones_h, sum_out, cnt_out, idx_a, idx_b, rows_a, rows_b,
             acc_sh, tab_sh, g_a, g_b, s_a, s_b,
             ones_v, zones_v, cnt_sh) = rest
        else:
            (sum_out, idx_a, idx_b, rows_a, rows_b,
             acc_sh, tab_sh, g_a, g_b, s_a, s_b) = rest
        cid = lax.axis_index("c")
        sid = lax.axis_index("s")
        wid = cid * _NS + sid
        idx = (idx_a, idx_b)
        rows = (rows_a, rows_b)
        gsem = (g_a, g_b)
        ssem = (s_a, s_b)

        pltpu.sync_copy(table.at[pl.ds(sid * _NPS, _NPS)],
                        tab_sh.at[pl.ds(sid * _NPS, _NPS)])
        pltpu.sync_copy(zrow, acc_sh.at[pl.ds(sid * _NPS, _NPS)])
        pltpu.sync_copy(zrow.at[pl.ds(0, _K * 128)], rows_b)
        if with_counts:
            pltpu.sync_copy(zcnt, cnt_sh.at[pl.ds(sid * _NPS, _NPS)])
            pltpu.sync_copy(ones_h, ones_v)
            pltpu.sync_copy(zcnt.at[pl.ds(0, 128)], zones_v)
        plsc.subcore_barrier()

        row_base = wid * _RPT

        def load_idx(buf, chunk):
            pltpu.sync_copy(edges.at[pl.ds(row_base + chunk * _K, _K)],
                            idx[buf])

        def gathers(buf, start):
            for j in range(_K):
                cp = pltpu.make_async_copy(
                    tab_sh.at[idx[buf].at[j, 0]],
                    rows[buf].at[pl.ds(j * 128, 128)], gsem[buf])
                if start:
                    cp.start()
                else:
                    cp.wait()

        def scatters(buf, start, prime=False):
            for j in range(_K):
                dsti = idx[buf].at[j, 1]
                cp = pltpu.make_async_copy(
                    rows[buf].at[pl.ds(j * 128, 128)],
                    acc_sh.at[dsti], ssem[buf])
                if start:
                    cp.start(add=True)
                else:
                    cp.wait()
                if with_counts:
                    csrc = zones_v if prime else ones_v
                    cp2 = pltpu.make_async_copy(csrc, cnt_sh.at[dsti],
                                                ssem[buf])
                    if start:
                        cp2.start(add=True)
                    else:
                        cp2.wait()

        load_idx(0, 0)
        gathers(0, start=True)
        load_idx(1, 0)
        scatters(1, start=True, prime=True)

        def pipelined(t, carry):
            i0 = 2 * t
            scatters(1, start=False)
            load_idx(1, i0 + 1)
            gathers(1, start=True)
            gathers(0, start=False)
            scatters(0, start=True)
            scatters(0, start=False)
            load_idx(0, i0 + 2)
            gathers(0, start=True)
            gathers(1, start=False)
            scatters(1, start=True)
            return carry

        lax.fori_loop(0, _T, pipelined, 0)
        gathers(0, start=False)
        scatters(1, start=False)
        plsc.subcore_barrier()

        pltpu.sync_copy(acc_sh.at[pl.ds(sid * _NPS, _NPS)],
                        sum_out.at[cid, pl.ds(sid * _NPS, _NPS)])
        if with_counts:
            pltpu.sync_copy(cnt_sh.at[pl.ds(sid * _NPS, _NPS)],
                            cnt_out.at[cid, pl.ds(sid * _NPS, _NPS)])

    return pl.kernel(
        body, mesh=mesh, out_type=out_type, scratch_types=scratch,
        compiler_params=pltpu.CompilerParams(use_tc_tiling_on_sc=False))


_sc_segsum_counts = _make_sc_segsum(True)
_sc_segsum_plain = _make_sc_segsum(False)


def _dense_body(relu, out_bf, p_ref, cnt_ref, x_ref, wl_ref, wr_ref, b_ref,
                o_ref):
    s = p_ref[0].astype(jnp.float32) + p_ref[1].astype(jnp.float32)
    c = cnt_ref[0, :, 0:1] + cnt_ref[1, :, 0:1]
    aggr = s / jnp.clip(c, 1.0, None)
    xf = x_ref[...].astype(jnp.float32)
    out = (jnp.dot(aggr, wl_ref[...], preferred_element_type=jnp.float32)
           + jnp.dot(xf, wr_ref[...], preferred_element_type=jnp.float32)
           + b_ref[...])
    if relu:
        out = jnp.maximum(out, 0.0)
    if out_bf:
        o_ref[...] = out.astype(_BF)
    else:
        o_ref[...] = out


def _dense(p3, cnt3, xbf, wl, wr, b, relu, out_bf):
    br = 1024
    grid = _NP // br
    odt = _BF if out_bf else jnp.float32
    return pl.pallas_call(
        functools.partial(_dense_body, relu, out_bf),
        grid=(grid,),
        in_specs=[
            pl.BlockSpec((_NC, br, _D), lambda i: (0, i, 0)),
            pl.BlockSpec((_NC, br, 16), lambda i: (0, i, 0)),
            pl.BlockSpec((br, _D), lambda i: (i, 0)),
            pl.BlockSpec((_D, _D), lambda i: (0, 0)),
            pl.BlockSpec((_D, _D), lambda i: (0, 0)),
            pl.BlockSpec((1, _D), lambda i: (0, 0)),
        ],
        out_specs=pl.BlockSpec((br, _D), lambda i: (i, 0)),
        out_shape=jax.ShapeDtypeStruct((_NP, _D), odt),
    )(p3, cnt3, xbf, wl, wr, b.reshape(1, _D))


def kernel(x, edge_index, W_l1, W_r1, b1, W_l2, W_r2, b2):
    er = edge_index.astype(jnp.int32).reshape(2, _E // 128, 128)
    er = er.transpose(1, 0, 2)
    pad_rows = jnp.broadcast_to(
        jnp.array([0, _NP - 1], jnp.int32)[None, :, None],
        (_EROWS_ST - _E // 128, 2, 128))
    edges = jnp.concatenate([er, pad_rows])
    zrow = jnp.zeros((_NPS, _D), _BF)
    zcnt = jnp.zeros((_NPS, 16), jnp.float32)
    ones = jnp.ones((128, 16), jnp.float32)

    xbf = jnp.pad(x.astype(_BF), ((0, _NP - _N), (0, 0)))
    s1, cnts = _sc_segsum_counts(xbf, edges, zrow, zcnt, ones)
    hbf = _dense(s1, cnts, xbf, W_l1, W_r1, b1, relu=True, out_bf=True)
    (s2,) = _sc_segsum_plain(hbf, edges, zrow)
    out = _dense(s2, cnts, hbf, W_l2, W_r2, b2, relu=False, out_bf=False)
    return out[:_N]
